# baseline (speedup 1.0000x reference)
import functools

import jax
import jax.numpy as jnp
from jax import lax
from jax.experimental import pallas as pl
from jax.experimental.pallas import tpu as pltpu
from jax.experimental.pallas import tpu_sc as plsc

_N_BLK = 8192
_SC_ROWS = 8192


def _tail_kernel(h_ref, w1_t_ref, ret_feat_ref, w_ret_t_ref, b_ret_ref,
                 w2_t_ref, b_lin_ref, batch_ref, out_ref, ret2_ref,
                 offs_ref, *, bsz):
    out_ref[...] = jnp.dot(h_ref[...], w1_t_ref[...],
                           preferred_element_type=jnp.float32)
    rm = jnp.mean(ret_feat_ref[...], axis=1)
    rp = jnp.dot(rm, w_ret_t_ref[...],
                 preferred_element_type=jnp.float32) + b_ret_ref[...]
    ret2_ref[...] = jnp.dot(rp, w2_t_ref[...],
                            preferred_element_type=jnp.float32) + b_lin_ref[...]
    bt = batch_ref[...]
    lane = lax.broadcasted_iota(jnp.int32, (1, bsz), 1)
    acc = jnp.zeros((1, bsz), jnp.int32)
    for b in range(bsz):
        s_b = jnp.sum((bt < b).astype(jnp.int32))
        acc = acc + jnp.where(lane == b, s_b, 0)
    offs_ref[...] = acc


def _head_kernel(batch_ref, h_ref, w1_t_ref, ret_feat_ref, w_ret_t_ref,
                 b_ret_ref, w2_t_ref, b_lin_ref, out_ref, *, nb, b):
    rm = jnp.mean(ret_feat_ref[...], axis=1)
    rp = jnp.dot(rm, w_ret_t_ref[...],
                 preferred_element_type=jnp.float32) + b_ret_ref[...]
    ret2 = jnp.dot(rp, w2_t_ref[...],
                   preferred_element_type=jnp.float32) + b_lin_ref[...]
    idx = batch_ref[0, :]
    oh = (idx[:, None] == lax.broadcasted_iota(jnp.int32, (nb, b), 1)
          ).astype(jnp.float32)
    out_ref[...] = (
        jnp.dot(h_ref[...], w1_t_ref[...],
                preferred_element_type=jnp.float32)
        + jnp.dot(oh, ret2, preferred_element_type=jnp.float32))


def kernel(h, ret_feat, batch, W_ret, b_ret, W_lin, b_lin):
    n, h_dim = h.shape
    bsz, r, ret_dim = ret_feat.shape
    w1_t = W_lin[:, :h_dim].T
    w2_t = W_lin[:, h_dim:].T

    sc_rows = _SC_ROWS
    tc_rows = n - sc_rows
    tc_blocks = tc_rows // _N_BLK
    zero2 = lambda i: (0, 0)

    hw1b, ret2, offs = pl.pallas_call(
        functools.partial(_tail_kernel, bsz=bsz),
        grid=(sc_rows // _N_BLK,),
        in_specs=[
            pl.BlockSpec((_N_BLK, h_dim), lambda i: (i + tc_blocks, 0)),
            pl.BlockSpec((h_dim, h_dim), zero2),
            pl.BlockSpec((bsz, r, ret_dim), lambda i: (0, 0, 0)),
            pl.BlockSpec((h_dim, h_dim), zero2),
            pl.BlockSpec((1, h_dim), zero2),
            pl.BlockSpec((h_dim, h_dim), zero2),
            pl.BlockSpec((1, h_dim), zero2),
            pl.BlockSpec((n // h_dim, h_dim), zero2),
        ],
        out_specs=[
            pl.BlockSpec((_N_BLK, h_dim), lambda i: (i, 0)),
            pl.BlockSpec((bsz, h_dim), zero2),
            pl.BlockSpec((1, bsz), zero2),
        ],
        out_shape=(jax.ShapeDtypeStruct((sc_rows, h_dim), jnp.float32),
                   jax.ShapeDtypeStruct((bsz, h_dim), jnp.float32),
                   jax.ShapeDtypeStruct((1, bsz), jnp.int32)),
    )(h, w1_t, ret_feat, W_ret.T, b_ret.reshape(1, h_dim), w2_t,
      b_lin.reshape(1, h_dim), batch.reshape(n // h_dim, h_dim))

    info = plsc.get_sparse_core_info()
    nw = info.num_cores * info.num_subcores
    chunk = sc_rows // nw
    cwords = chunk * h_dim
    mesh = plsc.VectorSubcoreMesh(core_axis_name="c", subcore_axis_name="s")

    @functools.partial(
        pl.kernel, mesh=mesh,
        out_type=jax.ShapeDtypeStruct((sc_rows * h_dim,), jnp.float32),
        scratch_types=[
            pltpu.VMEM((bsz * h_dim,), jnp.float32),
            pltpu.VMEM((16,), jnp.int32),
            pltpu.VMEM((cwords,), jnp.float32),
            pltpu.SemaphoreType.DMA,
            pltpu.SemaphoreType.DMA,
        ],
    )
    def _sc_add(hw1_hbm, offs_hbm, ret2_hbm, out_hbm, table_v, offs_v,
                hv, sin, sout):
        wid = lax.axis_index("s") * info.num_cores + lax.axis_index("c")
        wbase = wid * cwords
        in_h = pltpu.async_copy(hw1_hbm.at[pl.ds(wbase, cwords)], hv, sin)
        pltpu.sync_copy(ret2_hbm, table_v)
        pltpu.sync_copy(offs_hbm, offs_v)
        ov = offs_v[...]
        bounds = [ov[b] for b in range(bsz)]
        bounds.append(jnp.int32(n))
        in_h.wait()
        cbase = tc_rows + wid * chunk
        for b in range(bsz):
            lo = jnp.clip(bounds[b] - cbase, 0, chunk)
            hi = jnp.clip(bounds[b + 1] - cbase, 0, chunk)
            trow = [table_v[pl.ds(b * h_dim + j * 16, 16)]
                    for j in range(h_dim // 16)]

            def _body(row, carry, hv=hv, trow=trow):
                for j in range(h_dim // 16):
                    sl = pl.ds(row * h_dim + j * 16, 16)
                    hv[sl] = hv[sl] + trow[j]
                return carry

            lax.fori_loop(lo, hi, _body, 0)
        pltpu.async_copy(hv, out_hbm.at[pl.ds(wbase, cwords)], sout).wait()

    batch3 = batch.reshape(n // _N_BLK, 1, _N_BLK)
    out_a = pl.pallas_call(
        functools.partial(_head_kernel, nb=_N_BLK, b=bsz),
        grid=(tc_blocks,),
        in_specs=[
            pl.BlockSpec((None, 1, _N_BLK), lambda i: (i, 0, 0)),
            pl.BlockSpec((_N_BLK, h_dim), lambda i: (i, 0)),
            pl.BlockSpec((h_dim, h_dim), zero2),
            pl.BlockSpec((bsz, r, ret_dim), lambda i: (0, 0, 0)),
            pl.BlockSpec((h_dim, h_dim), zero2),
            pl.BlockSpec((1, h_dim), zero2),
            pl.BlockSpec((h_dim, h_dim), zero2),
            pl.BlockSpec((1, h_dim), zero2),
        ],
        out_specs=pl.BlockSpec((_N_BLK, h_dim), lambda i: (i, 0)),
        out_shape=jax.ShapeDtypeStruct((n, h_dim), jnp.float32),
    )(batch3, h, w1_t, ret_feat, W_ret.T, b_ret.reshape(1, h_dim), w2_t,
      b_lin.reshape(1, h_dim))

    out_b = _sc_add(hw1b.reshape(sc_rows * h_dim), offs.reshape(bsz),
                    ret2.reshape(bsz * h_dim))

    return lax.dynamic_update_slice(
        out_a, out_b.reshape(sc_rows, h_dim), (tc_rows, 0))


# speedup vs baseline: 1.0219x; 1.0219x over previous
import functools

import jax
import jax.numpy as jnp
from jax import lax
from jax.experimental import pallas as pl
from jax.experimental.pallas import tpu as pltpu
from jax.experimental.pallas import tpu_sc as plsc

_TAIL_BLK = 4096
_HEAD_BLK = 12288
_SC_ROWS = 8192


def _tail_kernel(h_ref, w1_t_ref, ret_feat_ref, w_ret_t_ref, b_ret_ref,
                 w2_t_ref, b_lin_ref, batch_ref, out_ref, ret2_ref,
                 offs_ref, *, bsz):
    out_ref[...] = jnp.dot(h_ref[...], w1_t_ref[...],
                           preferred_element_type=jnp.float32)
    rm = jnp.mean(ret_feat_ref[...], axis=1)
    rp = jnp.dot(rm, w_ret_t_ref[...],
                 preferred_element_type=jnp.float32) + b_ret_ref[...]
    ret2_ref[...] = jnp.dot(rp, w2_t_ref[...],
                            preferred_element_type=jnp.float32) + b_lin_ref[...]
    bt = batch_ref[...]
    lane = lax.broadcasted_iota(jnp.int32, (1, bsz), 1)
    acc = jnp.zeros((1, bsz), jnp.int32)
    for b in range(bsz):
        s_b = jnp.sum((bt < b).astype(jnp.int32))
        acc = acc + jnp.where(lane == b, s_b, 0)
    offs_ref[...] = acc


def _head_kernel(batch_ref, h_ref, w1_t_ref, ret_feat_ref, w_ret_t_ref,
                 b_ret_ref, w2_t_ref, b_lin_ref, out_ref, *, nb, b):
    rm = jnp.mean(ret_feat_ref[...], axis=1)
    rp = jnp.dot(rm, w_ret_t_ref[...],
                 preferred_element_type=jnp.float32) + b_ret_ref[...]
    ret2 = jnp.dot(rp, w2_t_ref[...],
                   preferred_element_type=jnp.float32) + b_lin_ref[...]
    idx = batch_ref[0, :]
    oh = (idx[:, None] == lax.broadcasted_iota(jnp.int32, (nb, b), 1)
          ).astype(jnp.float32)
    out_ref[...] = (
        jnp.dot(h_ref[...], w1_t_ref[...],
                preferred_element_type=jnp.float32)
        + jnp.dot(oh, ret2, preferred_element_type=jnp.float32))


def kernel(h, ret_feat, batch, W_ret, b_ret, W_lin, b_lin):
    n, h_dim = h.shape
    bsz, r, ret_dim = ret_feat.shape
    w1_t = W_lin[:, :h_dim].T
    w2_t = W_lin[:, h_dim:].T

    sc_rows = _SC_ROWS
    tc_rows = n - sc_rows
    zero2 = lambda i: (0, 0)

    hw1b, ret2, offs = pl.pallas_call(
        functools.partial(_tail_kernel, bsz=bsz),
        grid=(sc_rows // _TAIL_BLK,),
        in_specs=[
            pl.BlockSpec((_TAIL_BLK, h_dim),
                         lambda i: (i + tc_rows // _TAIL_BLK, 0)),
            pl.BlockSpec((h_dim, h_dim), zero2),
            pl.BlockSpec((bsz, r, ret_dim), lambda i: (0, 0, 0)),
            pl.BlockSpec((h_dim, h_dim), zero2),
            pl.BlockSpec((1, h_dim), zero2),
            pl.BlockSpec((h_dim, h_dim), zero2),
            pl.BlockSpec((1, h_dim), zero2),
            pl.BlockSpec((n // h_dim, h_dim), zero2),
        ],
        out_specs=[
            pl.BlockSpec((_TAIL_BLK, h_dim), lambda i: (i, 0)),
            pl.BlockSpec((bsz, h_dim), zero2),
            pl.BlockSpec((1, bsz), zero2),
        ],
        out_shape=(jax.ShapeDtypeStruct((sc_rows, h_dim), jnp.float32),
                   jax.ShapeDtypeStruct((bsz, h_dim), jnp.float32),
                   jax.ShapeDtypeStruct((1, bsz), jnp.int32)),
    )(h, w1_t, ret_feat, W_ret.T, b_ret.reshape(1, h_dim), w2_t,
      b_lin.reshape(1, h_dim), batch.reshape(n // h_dim, h_dim))

    info = plsc.get_sparse_core_info()
    nw = info.num_cores * info.num_subcores
    chunk = sc_rows // nw
    cwords = chunk * h_dim
    mesh = plsc.VectorSubcoreMesh(core_axis_name="c", subcore_axis_name="s")

    @functools.partial(
        pl.kernel, mesh=mesh,
        out_type=jax.ShapeDtypeStruct((sc_rows * h_dim,), jnp.float32),
        scratch_types=[
            pltpu.VMEM((bsz * h_dim,), jnp.float32),
            pltpu.VMEM((16,), jnp.int32),
            pltpu.VMEM((cwords,), jnp.float32),
            pltpu.SemaphoreType.DMA,
            pltpu.SemaphoreType.DMA,
        ],
    )
    def _sc_add(hw1_hbm, offs_hbm, ret2_hbm, out_hbm, table_v, offs_v,
                hv, sin, sout):
        wid = lax.axis_index("s") * info.num_cores + lax.axis_index("c")
        wbase = wid * cwords
        in_h = pltpu.async_copy(hw1_hbm.at[pl.ds(wbase, cwords)], hv, sin)
        pltpu.sync_copy(ret2_hbm, table_v)
        pltpu.sync_copy(offs_hbm, offs_v)
        ov = offs_v[...]
        bounds = [ov[b] for b in range(bsz)]
        bounds.append(jnp.int32(n))
        in_h.wait()
        cbase = tc_rows + wid * chunk
        for b in range(bsz):
            lo = jnp.clip(bounds[b] - cbase, 0, chunk)
            hi = jnp.clip(bounds[b + 1] - cbase, 0, chunk)
            trow = [table_v[pl.ds(b * h_dim + j * 16, 16)]
                    for j in range(h_dim // 16)]

            def _body(row, carry, hv=hv, trow=trow):
                for j in range(h_dim // 16):
                    sl = pl.ds(row * h_dim + j * 16, 16)
                    hv[sl] = hv[sl] + trow[j]
                return carry

            lax.fori_loop(lo, hi, _body, 0)
        pltpu.async_copy(hv, out_hbm.at[pl.ds(wbase, cwords)], sout).wait()

    batch3 = batch[:tc_rows].reshape(tc_rows // _HEAD_BLK, 1, _HEAD_BLK)
    out_a = pl.pallas_call(
        functools.partial(_head_kernel, nb=_HEAD_BLK, b=bsz),
        grid=(tc_rows // _HEAD_BLK,),
        in_specs=[
            pl.BlockSpec((None, 1, _HEAD_BLK), lambda i: (i, 0, 0)),
            pl.BlockSpec((_HEAD_BLK, h_dim), lambda i: (i, 0)),
            pl.BlockSpec((h_dim, h_dim), zero2),
            pl.BlockSpec((bsz, r, ret_dim), lambda i: (0, 0, 0)),
            pl.BlockSpec((h_dim, h_dim), zero2),
            pl.BlockSpec((1, h_dim), zero2),
            pl.BlockSpec((h_dim, h_dim), zero2),
            pl.BlockSpec((1, h_dim), zero2),
        ],
        out_specs=pl.BlockSpec((_HEAD_BLK, h_dim), lambda i: (i, 0)),
        out_shape=jax.ShapeDtypeStruct((n, h_dim), jnp.float32),
    )(batch3, h, w1_t, ret_feat, W_ret.T, b_ret.reshape(1, h_dim), w2_t,
      b_lin.reshape(1, h_dim))

    out_b = _sc_add(hw1b.reshape(sc_rows * h_dim), offs.reshape(bsz),
                    ret2.reshape(bsz * h_dim))

    return lax.dynamic_update_slice(
        out_a, out_b.reshape(sc_rows, h_dim), (tc_rows, 0))
